# baseline (speedup 1.0000x reference)
import functools

import jax
import jax.numpy as jnp
from jax import lax
from jax.experimental import pallas as pl
from jax.experimental.pallas import tpu as pltpu
from jax.experimental.pallas import tpu_sc as plsc

IN_FEAT = 128
OUT_FEAT = 128
NUM_RELS = 8
NUM_BASES = 4
N_NODES = 10000
N_EDGES = 320000

NC = 2
NS = 16
L = 16
NW = NC * NS
CH = 128
RPW = 80
EROWS = 2568
SINK = N_NODES
HPAD = 10240
RPT = HPAD // NS
ZROWS = 32
RB = 1000


def _embed_matmul(S, w2):
    def body(s_ref, w_ref, o_ref):
        o_ref[...] = jnp.dot(s_ref[...], w_ref[...],
                             preferred_element_type=jnp.float32)

    return pl.pallas_call(
        body,
        out_shape=jax.ShapeDtypeStruct((NUM_RELS * IN_FEAT, OUT_FEAT),
                                       jnp.float32),
    )(S, w2)


def _sc_scatter(src, dst, rel, feat, embed, zeros_h):
    mesh = plsc.VectorSubcoreMesh(core_axis_name="c", subcore_axis_name="s")

    idx_t = pltpu.VMEM((CH,), jnp.int32)
    rows_t = pltpu.VMEM((CH, OUT_FEAT), jnp.float32)

    @functools.partial(
        pl.kernel,
        mesh=mesh,
        out_type=jax.ShapeDtypeStruct((NC, HPAD, OUT_FEAT), jnp.float32),
        scratch_types=(
            [idx_t] * 4 +
            [idx_t] * 4 +
            [idx_t] * 4 +
            [idx_t] * 4 +
            [rows_t] * 2 +
            [pltpu.VMEM((ZROWS, OUT_FEAT), jnp.float32),
             pltpu.VMEM_SHARED((HPAD, OUT_FEAT), jnp.float32)] +
            [pltpu.SemaphoreType.DMA] * 11
        ),
    )
    def k(src_h, dst_h, rel_h, feat_h, embed_h, zeros_hbm, out_h,
          *refs):
        src_v = refs[0:4]
        dst_v = refs[4:8]
        rel_v = refs[8:12]
        idx_v = refs[12:16]
        rows_v = refs[16:18]
        zbuf = refs[18]
        h_sh = refs[19]
        sem_e = refs[20:24]
        sem_f = refs[24:28]
        sem_s = refs[28:30]
        sem_g = refs[30]
        c = lax.axis_index("c")
        s = lax.axis_index("s")
        wid = s * NC + c
        e_base = RPW * CH * wid

        def edata_issue(j, u):
            e0 = e_base + j * CH
            pltpu.async_copy(src_h.at[pl.ds(e0, CH)], src_v[u], sem_e[u])
            pltpu.async_copy(dst_h.at[pl.ds(e0, CH)], dst_v[u], sem_e[u])
            pltpu.async_copy(rel_h.at[pl.ds(e0, CH)], rel_v[u], sem_e[u])

        def edata_wait(u):
            for r in (src_v, dst_v, rel_v):
                pltpu.make_async_copy(src_h.at[pl.ds(0, CH)], r[u],
                                      sem_e[u]).wait()

        def fgather_issue(u):
            pltpu.async_copy(feat_h.at[src_v[u]], idx_v[u], sem_f[u])

        def fgather_wait(u):
            pltpu.make_async_copy(feat_h.at[src_v[u]], idx_v[u],
                                  sem_f[u]).wait()

        def scatter_wait(p):
            pltpu.make_async_copy(rows_v[p], h_sh.at[dst_v[p]],
                                  sem_s[p]).wait()

        pltpu.sync_copy(zeros_hbm, zbuf)
        arow0 = s * RPT

        def zfill(q, carry):
            pltpu.sync_copy(zbuf, h_sh.at[pl.ds(arow0 + q * ZROWS, ZROWS)])
            return carry

        lax.fori_loop(0, RPT // ZROWS, zfill, 0)
        plsc.subcore_barrier()

        def block(j, carry):
            edata_issue(j, 0)
            edata_wait(0)
            fgather_issue(0)
            fgather_wait(0)
            for g in range(CH // L):
                sl = pl.ds(g * L, L)
                idx_v[0][sl] = rel_v[0][sl] * IN_FEAT + idx_v[0][sl]
            pltpu.async_copy(embed_h.at[idx_v[0]], rows_v[0], sem_g).wait()
            pltpu.async_copy(rows_v[0], h_sh.at[dst_v[0]], sem_s[0],
                             add=True).wait()
            return carry

        lax.fori_loop(0, RPW, block, 0)
        plsc.subcore_barrier()

        def outq(q, carry):
            r0 = arow0 + q * ZROWS
            pltpu.sync_copy(h_sh.at[pl.ds(r0, ZROWS)], zbuf)
            pltpu.sync_copy(zbuf, out_h.at[c, pl.ds(r0, ZROWS)])
            return carry

        lax.fori_loop(0, RPT // ZROWS, outq, 0)

    return k(src, dst, rel, feat, embed, zeros_h)


def _reduce(partials):
    def body(p_ref, hsum_ref, stats_ref):
        i = pl.program_id(0)
        sblk = p_ref[0] + p_ref[1]
        hsum_ref[...] = sblk
        part = jnp.concatenate(
            [jnp.sum(sblk, axis=0, keepdims=True),
             jnp.sum(sblk * sblk, axis=0, keepdims=True),
             jnp.zeros((6, OUT_FEAT), jnp.float32)], axis=0)

        @pl.when(i == 0)
        def _():
            stats_ref[...] = jnp.zeros((8, OUT_FEAT), jnp.float32)

        stats_ref[...] += part

    return pl.pallas_call(
        body,
        grid=(N_NODES // RB,),
        in_specs=[pl.BlockSpec((NC, RB, OUT_FEAT), lambda i: (0, i, 0))],
        out_specs=[pl.BlockSpec((RB, OUT_FEAT), lambda i: (i, 0)),
                   pl.BlockSpec((8, OUT_FEAT), lambda i: (0, 0))],
        out_shape=[jax.ShapeDtypeStruct((N_NODES, OUT_FEAT), jnp.float32),
                   jax.ShapeDtypeStruct((8, OUT_FEAT), jnp.float32)],
    )(partials)


def _bn(hsum, stats, gamma, beta):
    def body(h_ref, st_ref, g_ref, b_ref, o_ref):
        mean = st_ref[0:1] * (1.0 / N_NODES)
        ex2 = st_ref[1:2] * (1.0 / N_NODES)
        var = ex2 - mean * mean
        inv = lax.rsqrt(var + 1e-5)
        o_ref[...] = (h_ref[...] - mean) * inv * g_ref[...] + b_ref[...]

    return pl.pallas_call(
        body,
        grid=(N_NODES // RB,),
        in_specs=[pl.BlockSpec((RB, OUT_FEAT), lambda i: (i, 0)),
                  pl.BlockSpec((8, OUT_FEAT), lambda i: (0, 0)),
                  pl.BlockSpec((1, OUT_FEAT), lambda i: (0, 0)),
                  pl.BlockSpec((1, OUT_FEAT), lambda i: (0, 0))],
        out_specs=pl.BlockSpec((RB, OUT_FEAT), lambda i: (i, 0)),
        out_shape=jax.ShapeDtypeStruct((N_NODES, OUT_FEAT), jnp.float32),
    )(hsum, stats, gamma, beta)


def kernel(features, edge_index, rel_type, weight, w_comp, bn_gamma, bn_beta):
    feat = features.astype(jnp.int32)
    npad = EROWS * CH - N_EDGES
    src = jnp.concatenate(
        [edge_index[0].astype(jnp.int32), jnp.zeros((npad,), jnp.int32)])
    dst = jnp.concatenate(
        [edge_index[1].astype(jnp.int32),
         SINK + (jnp.arange(npad, dtype=jnp.int32) % (HPAD - N_NODES))])
    rel = jnp.concatenate(
        [rel_type.astype(jnp.int32), jnp.zeros((npad,), jnp.int32)])

    k = jnp.arange(NUM_RELS * IN_FEAT)
    r = k // IN_FEAT
    f = k % IN_FEAT
    i = 16 * r + f // 8
    j = f % 8
    S = jnp.zeros((NUM_RELS * IN_FEAT, NUM_BASES * IN_FEAT), jnp.float32)
    cols = i[:, None] * NUM_BASES + jnp.arange(NUM_BASES)[None, :]
    S = S.at[k[:, None], cols].set(w_comp[j])

    embed = _embed_matmul(
        S, weight.reshape(NUM_BASES * IN_FEAT, OUT_FEAT).astype(jnp.float32))

    zeros_h = jnp.zeros((ZROWS, OUT_FEAT), jnp.float32)
    partials = _sc_scatter(src, dst, rel, feat, embed, zeros_h)
    hsum, stats = _reduce(partials)
    return _bn(hsum, stats, bn_gamma.reshape(1, OUT_FEAT),
               bn_beta.reshape(1, OUT_FEAT))


# speedup vs baseline: 1.1876x; 1.1876x over previous
import functools

import jax
import jax.numpy as jnp
from jax import lax
from jax.experimental import pallas as pl
from jax.experimental.pallas import tpu as pltpu
from jax.experimental.pallas import tpu_sc as plsc

IN_FEAT = 128
OUT_FEAT = 128
NUM_RELS = 8
NUM_BASES = 4
N_NODES = 10000
N_EDGES = 320000

NC = 2
NS = 16
L = 16
NW = NC * NS
CH = 64
RPW = 160
EPW = RPW * CH
SINK = N_NODES
HPAD = 10240
RPT = HPAD // NS
RB = 1000


def _embed_matmul(S, w2):
    def body(s_ref, w_ref, o_ref):
        o_ref[...] = jnp.dot(s_ref[...], w_ref[...],
                             preferred_element_type=jnp.float32)

    return pl.pallas_call(
        body,
        out_shape=jax.ShapeDtypeStruct((NUM_RELS * IN_FEAT, OUT_FEAT),
                                       jnp.float32),
    )(S, w2)


def _sc_scatter(src, dst, relb, feat, embed, zeros_h):
    mesh = plsc.VectorSubcoreMesh(core_axis_name="c", subcore_axis_name="s")

    edges_t = pltpu.VMEM((EPW,), jnp.int32)
    rows_t = pltpu.VMEM((CH, OUT_FEAT), jnp.float32)

    @functools.partial(
        pl.kernel,
        mesh=mesh,
        out_type=jax.ShapeDtypeStruct((NC, HPAD, OUT_FEAT), jnp.float32),
        scratch_types=(
            [edges_t] * 3 +
            [rows_t] * 2 +
            [pltpu.VMEM_SHARED((HPAD, OUT_FEAT), jnp.float32)] +
            [pltpu.SemaphoreType.DMA] * 7
        ),
    )
    def k(src_h, dst_h, relb_h, feat_h, embed_h, zeros_hbm, out_h,
          *refs):
        src_v, dst_v, idx_v = refs[0:3]
        rows_v = refs[3:5]
        h_sh = refs[5]
        sem_e = refs[6]
        sem_f = refs[7]
        sem_z = refs[8]
        sem_g = refs[9:11]
        sem_s = refs[11:13]
        c = lax.axis_index("c")
        s = lax.axis_index("s")
        wid = s * NC + c
        e_base = EPW * wid

        def gather_issue(j, p):
            pltpu.async_copy(embed_h.at[idx_v.at[pl.ds(j * CH, CH)]],
                             rows_v[p], sem_g[p])

        def gather_wait(p):
            pltpu.make_async_copy(embed_h.at[idx_v.at[pl.ds(0, CH)]],
                                  rows_v[p], sem_g[p]).wait()

        def scatter_issue(j, p):
            pltpu.async_copy(rows_v[p], h_sh.at[dst_v.at[pl.ds(j * CH, CH)]],
                             sem_s[p], add=True)

        def scatter_wait(p):
            pltpu.make_async_copy(rows_v[p], h_sh.at[pl.ds(0, CH)],
                                  sem_s[p]).wait()

        esl = pl.ds(e_base, EPW)
        arow0 = s * RPT
        pltpu.async_copy(src_h.at[esl], src_v, sem_e)
        pltpu.async_copy(dst_h.at[esl], dst_v, sem_e)
        pltpu.async_copy(zeros_hbm, h_sh.at[pl.ds(arow0, RPT)], sem_z)
        for _ in range(2):
            pltpu.make_async_copy(src_h.at[esl], src_v, sem_e).wait()

        pltpu.async_copy(feat_h.at[src_v], idx_v, sem_f).wait()
        pltpu.async_copy(relb_h.at[esl], src_v, sem_e).wait()

        def bidx(q, carry):
            sl = pl.ds(q * L, L)
            idx_v[sl] = src_v[sl] + idx_v[sl]
            return carry

        lax.fori_loop(0, EPW // L, bidx, 0)
        pltpu.make_async_copy(zeros_hbm, h_sh.at[pl.ds(0, RPT)],
                              sem_z).wait()
        plsc.subcore_barrier()

        gather_issue(0, 0)

        def block(j, p):
            @pl.when(j >= 1)
            def _():
                scatter_wait(1 - p)

            jn = jnp.minimum(j + 1, RPW - 1)

            @pl.when(j < RPW - 1)
            def _():
                gather_issue(jn, 1 - p)

            gather_wait(p)
            scatter_issue(j, p)

        def step(t, carry):
            block(2 * t, 0)
            block(2 * t + 1, 1)
            return carry

        lax.fori_loop(0, RPW // 2, step, 0)
        scatter_wait((RPW - 1) % 2)
        plsc.subcore_barrier()

        pltpu.async_copy(h_sh.at[pl.ds(arow0, RPT)],
                         out_h.at[c, pl.ds(arow0, RPT)], sem_z).wait()

    return k(src, dst, relb, feat, embed, zeros_h)


def _reduce(partials):
    def body(p_ref, hsum_ref, stats_ref):
        i = pl.program_id(0)
        sblk = p_ref[0] + p_ref[1]
        hsum_ref[...] = sblk
        part = jnp.concatenate(
            [jnp.sum(sblk, axis=0, keepdims=True),
             jnp.sum(sblk * sblk, axis=0, keepdims=True),
             jnp.zeros((6, OUT_FEAT), jnp.float32)], axis=0)

        @pl.when(i == 0)
        def _():
            stats_ref[...] = jnp.zeros((8, OUT_FEAT), jnp.float32)

        stats_ref[...] += part

    return pl.pallas_call(
        body,
        grid=(N_NODES // RB,),
        in_specs=[pl.BlockSpec((NC, RB, OUT_FEAT), lambda i: (0, i, 0))],
        out_specs=[pl.BlockSpec((RB, OUT_FEAT), lambda i: (i, 0)),
                   pl.BlockSpec((8, OUT_FEAT), lambda i: (0, 0))],
        out_shape=[jax.ShapeDtypeStruct((N_NODES, OUT_FEAT), jnp.float32),
                   jax.ShapeDtypeStruct((8, OUT_FEAT), jnp.float32)],
    )(partials)


def _bn(hsum, stats, gamma, beta):
    def body(h_ref, st_ref, g_ref, b_ref, o_ref):
        mean = st_ref[0:1] * (1.0 / N_NODES)
        ex2 = st_ref[1:2] * (1.0 / N_NODES)
        var = ex2 - mean * mean
        inv = lax.rsqrt(var + 1e-5)
        o_ref[...] = (h_ref[...] - mean) * inv * g_ref[...] + b_ref[...]

    return pl.pallas_call(
        body,
        grid=(N_NODES // RB,),
        in_specs=[pl.BlockSpec((RB, OUT_FEAT), lambda i: (i, 0)),
                  pl.BlockSpec((8, OUT_FEAT), lambda i: (0, 0)),
                  pl.BlockSpec((1, OUT_FEAT), lambda i: (0, 0)),
                  pl.BlockSpec((1, OUT_FEAT), lambda i: (0, 0))],
        out_specs=pl.BlockSpec((RB, OUT_FEAT), lambda i: (i, 0)),
        out_shape=jax.ShapeDtypeStruct((N_NODES, OUT_FEAT), jnp.float32),
    )(hsum, stats, gamma, beta)


def kernel(features, edge_index, rel_type, weight, w_comp, bn_gamma, bn_beta):
    feat = features.astype(jnp.int32)
    npad = NW * EPW - N_EDGES
    src = jnp.concatenate(
        [edge_index[0].astype(jnp.int32), jnp.zeros((npad,), jnp.int32)])
    dst = jnp.concatenate(
        [edge_index[1].astype(jnp.int32),
         SINK + (jnp.arange(npad, dtype=jnp.int32) % (HPAD - N_NODES))])
    relb = jnp.concatenate(
        [rel_type.astype(jnp.int32) * IN_FEAT, jnp.zeros((npad,), jnp.int32)])

    k = jnp.arange(NUM_RELS * IN_FEAT)
    r = k // IN_FEAT
    f = k % IN_FEAT
    i = 16 * r + f // 8
    j = f % 8
    S = jnp.zeros((NUM_RELS * IN_FEAT, NUM_BASES * IN_FEAT), jnp.float32)
    cols = i[:, None] * NUM_BASES + jnp.arange(NUM_BASES)[None, :]
    S = S.at[k[:, None], cols].set(w_comp[j])

    embed = _embed_matmul(
        S, weight.reshape(NUM_BASES * IN_FEAT, OUT_FEAT).astype(jnp.float32))

    zeros_h = jnp.zeros((RPT, OUT_FEAT), jnp.float32)
    partials = _sc_scatter(src, dst, relb, feat, embed, zeros_h)
    hsum, stats = _reduce(partials)
    return _bn(hsum, stats, bn_gamma.reshape(1, OUT_FEAT),
               bn_beta.reshape(1, OUT_FEAT))


# speedup vs baseline: 2.6011x; 2.1903x over previous
import functools

import jax
import jax.numpy as jnp
from jax import lax
from jax.experimental import pallas as pl
from jax.experimental.pallas import tpu as pltpu
from jax.experimental.pallas import tpu_sc as plsc

IN_FEAT = 128
OUT_FEAT = 128
NUM_RELS = 8
NUM_BASES = 4
N_NODES = 10000
N_EDGES = 320000

NC = 2
NS = 16
L = 16
CW = NUM_RELS * IN_FEAT
ET = N_EDGES // NS
VECS = ET // L
NBLK = 10
BPC = NBLK // NC
RPB = 1040
HROWS = NBLK * RPB
ZR = (RPB + L) // NS
RW = RPB // NS
RBM = 520
CHK = 2000
NCHK = ET // CHK


def _embed_matmul(S, w2):
    def body(s_ref, w_ref, o_ref):
        o_ref[...] = jnp.dot(s_ref[...], w_ref[...],
                             preferred_element_type=jnp.float32)

    return pl.pallas_call(
        body,
        out_shape=jax.ShapeDtypeStruct((NUM_RELS * IN_FEAT, OUT_FEAT),
                                       jnp.float32),
    )(S, w2)


def _sc_counts(src, base, feat, ones_hbm, zeros_h):
    mesh = plsc.VectorSubcoreMesh(core_axis_name="c", subcore_axis_name="s")

    ibuf = pltpu.VMEM((ET,), jnp.int32)

    @functools.partial(
        pl.kernel,
        mesh=mesh,
        out_type=jax.ShapeDtypeStruct((HROWS * CW,), jnp.float32),
        scratch_types=(
            [ibuf] * 3 +
            [pltpu.VMEM((CHK,), jnp.float32)] +
            [pltpu.VMEM_SHARED(((RPB + L) * CW,), jnp.float32)] +
            [pltpu.SemaphoreType.DMA] * 4
        ),
    )
    def k(src_h, base_h, feat_h, ones_h, zeros_hbm, out_h, *refs):
        av, fidxv, scv = refs[0:3]
        onesv = refs[3]
        cblk = refs[4]
        sem_e, sem_f, sem_z, sem_s = refs[5:9]
        c = lax.axis_index("c")
        s = lax.axis_index("s")
        esl = pl.ds(ET * s, ET)

        pltpu.async_copy(src_h.at[esl], av, sem_e)
        pltpu.async_copy(base_h.at[esl], fidxv, sem_e)
        pltpu.async_copy(ones_h, onesv, sem_e)
        pltpu.make_async_copy(src_h.at[esl], av, sem_e).wait()
        pltpu.make_async_copy(base_h.at[esl], fidxv, sem_e).wait()
        pltpu.make_async_copy(ones_h, onesv, sem_e).wait()
        pltpu.async_copy(feat_h.at[av], scv, sem_f).wait()

        def prep(q, carry):
            sl = pl.ds(q * L, L)
            fidxv[sl] = fidxv[sl] + scv[sl]
            return carry

        lax.fori_loop(0, VECS, prep, 0)

        for p in range(BPC):
            lo = (c * BPC + p) * RPB * CW

            zsl = pl.ds(s * ZR * CW, ZR * CW)
            pltpu.async_copy(zeros_hbm, cblk.at[zsl], sem_z)

            def sidx(q, carry):
                sl = pl.ds(q * L, L)
                t = fidxv[sl] - lo
                sink = RPB * CW + (fidxv[sl] & (CW - 1))
                scv[sl] = jnp.where((t >= 0) & (t < RPB * CW), t, sink)
                return carry

            lax.fori_loop(0, VECS, sidx, 0)
            pltpu.make_async_copy(zeros_hbm, cblk.at[pl.ds(0, ZR * CW)],
                                  sem_z).wait()
            plsc.subcore_barrier()

            for k2 in range(NCHK):
                pltpu.async_copy(
                    onesv, cblk.at[scv.at[pl.ds(k2 * CHK, CHK)]],
                    sem_s, add=True)
                if k2 >= 3:
                    pltpu.make_async_copy(onesv, cblk.at[pl.ds(0, CHK)],
                                          sem_s).wait()
            for _ in range(3):
                pltpu.make_async_copy(onesv, cblk.at[pl.ds(0, CHK)],
                                      sem_s).wait()
            plsc.subcore_barrier()

            wsl = pl.ds(s * RW * CW, RW * CW)
            hsl = pl.ds(lo + s * RW * CW, RW * CW)
            pltpu.async_copy(cblk.at[wsl], out_h.at[hsl], sem_z)
            pltpu.make_async_copy(cblk.at[wsl], out_h.at[hsl], sem_z).wait()
            plsc.subcore_barrier()

    return k(src, base, feat, ones_hbm, zeros_h)


def _matmul_stats(Cm, embed):
    def body(c_ref, e_ref, h_ref, st_ref):
        i = pl.program_id(0)
        h = jnp.dot(c_ref[...], e_ref[...],
                    preferred_element_type=jnp.float32)
        h_ref[...] = h
        part = jnp.concatenate(
            [jnp.sum(h, axis=0, keepdims=True),
             jnp.sum(h * h, axis=0, keepdims=True),
             jnp.zeros((6, OUT_FEAT), jnp.float32)], axis=0)

        @pl.when(i == 0)
        def _():
            st_ref[...] = jnp.zeros((8, OUT_FEAT), jnp.float32)

        st_ref[...] += part

    return pl.pallas_call(
        body,
        grid=(HROWS // RBM,),
        in_specs=[pl.BlockSpec((RBM, CW), lambda i: (i, 0)),
                  pl.BlockSpec((CW, OUT_FEAT), lambda i: (0, 0))],
        out_specs=[pl.BlockSpec((RBM, OUT_FEAT), lambda i: (i, 0)),
                   pl.BlockSpec((8, OUT_FEAT), lambda i: (0, 0))],
        out_shape=[jax.ShapeDtypeStruct((HROWS, OUT_FEAT), jnp.float32),
                   jax.ShapeDtypeStruct((8, OUT_FEAT), jnp.float32)],
    )(Cm, embed)


def _bn(hsum, stats, gamma, beta):
    def body(h_ref, st_ref, g_ref, b_ref, o_ref):
        mean = st_ref[0:1] * (1.0 / N_NODES)
        ex2 = st_ref[1:2] * (1.0 / N_NODES)
        var = ex2 - mean * mean
        inv = lax.rsqrt(var + 1e-5)
        o_ref[...] = (h_ref[...] - mean) * inv * g_ref[...] + b_ref[...]

    return pl.pallas_call(
        body,
        grid=(HROWS // RBM,),
        in_specs=[pl.BlockSpec((RBM, OUT_FEAT), lambda i: (i, 0)),
                  pl.BlockSpec((8, OUT_FEAT), lambda i: (0, 0)),
                  pl.BlockSpec((1, OUT_FEAT), lambda i: (0, 0)),
                  pl.BlockSpec((1, OUT_FEAT), lambda i: (0, 0))],
        out_specs=pl.BlockSpec((RBM, OUT_FEAT), lambda i: (i, 0)),
        out_shape=jax.ShapeDtypeStruct((HROWS, OUT_FEAT), jnp.float32),
    )(hsum, stats, gamma, beta)


def kernel(features, edge_index, rel_type, weight, w_comp, bn_gamma, bn_beta):
    feat = features.astype(jnp.int32)
    src = edge_index[0].astype(jnp.int32)
    base = (edge_index[1].astype(jnp.int32) * CW
            + rel_type.astype(jnp.int32) * IN_FEAT)

    k = jnp.arange(NUM_RELS * IN_FEAT)
    r = k // IN_FEAT
    f = k % IN_FEAT
    i = 16 * r + f // 8
    j = f % 8
    S = jnp.zeros((NUM_RELS * IN_FEAT, NUM_BASES * IN_FEAT), jnp.float32)
    cols = i[:, None] * NUM_BASES + jnp.arange(NUM_BASES)[None, :]
    S = S.at[k[:, None], cols].set(w_comp[j])

    embed = _embed_matmul(
        S, weight.reshape(NUM_BASES * IN_FEAT, OUT_FEAT).astype(jnp.float32))

    ones_hbm = jnp.ones((CHK,), jnp.float32)
    zeros_h = jnp.zeros((ZR * CW,), jnp.float32)
    counts = _sc_counts(src, base, feat, ones_hbm, zeros_h)
    hsum, stats = _matmul_stats(counts.reshape(HROWS, CW), embed)
    h = _bn(hsum, stats, bn_gamma.reshape(1, OUT_FEAT),
            bn_beta.reshape(1, OUT_FEAT))
    return h[:N_NODES]
